# baseline (speedup 1.0000x reference)
import jax
import jax.numpy as jnp
from jax.experimental import pallas as pl

_ROWS, _COLS = 128, 32768
_BLOCK_ROWS = 16


def _onehot_rank_body(x_ref, o_ref):
    x = x_ref[...]
    pivot = x[:, 0:1]
    gt = (x > pivot).astype(jnp.int32)
    cnt = jnp.sum(gt, axis=1, keepdims=True)
    iota = jax.lax.broadcasted_iota(jnp.int32, x.shape, 1)
    o_ref[...] = (iota == cnt).astype(jnp.float32)


def kernel(scores):
    return pl.pallas_call(
        _onehot_rank_body,
        grid=(_ROWS // _BLOCK_ROWS,),
        in_specs=[pl.BlockSpec((_BLOCK_ROWS, _COLS), lambda i: (i, 0))],
        out_specs=pl.BlockSpec((_BLOCK_ROWS, _COLS), lambda i: (i, 0)),
        out_shape=jax.ShapeDtypeStruct((_ROWS, _COLS), jnp.float32),
    )(scores)


# speedup vs baseline: 1.2746x; 1.2746x over previous
import jax
import jax.numpy as jnp
from jax.experimental import pallas as pl

_ROWS, _COLS = 128, 32768
_BLOCK_ROWS = 64


def _onehot_rank_body(x_ref, o_ref):
    x = x_ref[...]
    pivot = x[:, 0:1]
    gt = (x > pivot).astype(jnp.int32)
    cnt = jnp.sum(gt, axis=1, keepdims=True)
    iota = jax.lax.broadcasted_iota(jnp.int32, x.shape, 1)
    o_ref[...] = (iota == cnt).astype(jnp.float32)


def kernel(scores):
    return pl.pallas_call(
        _onehot_rank_body,
        grid=(_ROWS // _BLOCK_ROWS,),
        in_specs=[pl.BlockSpec((_BLOCK_ROWS, _COLS), lambda i: (i, 0))],
        out_specs=pl.BlockSpec((_BLOCK_ROWS, _COLS), lambda i: (i, 0)),
        out_shape=jax.ShapeDtypeStruct((_ROWS, _COLS), jnp.float32),
    )(scores)
